# Initial kernel scaffold; baseline (speedup 1.0000x reference)
import jax
import jax.numpy as jnp
from jax.experimental import pallas as pl


def kernel(x):
    raise NotImplementedError("write your pallas kernel here")


# speedup vs baseline: 3.0562x; 3.0562x over previous
import functools

import jax
import jax.numpy as jnp
from jax import lax
from jax.experimental import pallas as pl
from jax.experimental.pallas import tpu as pltpu
from jax.experimental.pallas import tpu_sc as plsc

_PROB = 0.12


def _jitter_index(l):
    k1, k2 = jax.random.split(jax.random.key(42))
    index = jnp.arange(l, dtype=jnp.float32)
    change = jax.random.bernoulli(k1, _PROB * 2.0, (l,)).astype(jnp.float32)
    shift = jax.random.bernoulli(k2, 0.5, (l,)).astype(jnp.float32) * 2.0 - 1.0
    index = index + change * shift
    index = jnp.clip(index.astype(jnp.int32), 0, l - 1)
    return index


def _sc_row_gather(x2, gidx, chunk):
    B, D = x2.shape
    info = plsc.get_sparse_core_info()
    nc, ns = info.num_cores, info.num_subcores
    nw = nc * ns
    rpw = B // nw
    n_chunks = rpw // chunk
    assert rpw % chunk == 0 and chunk % 8 == 0

    mesh = plsc.VectorSubcoreMesh(core_axis_name="c", subcore_axis_name="s")

    @functools.partial(
        pl.kernel,
        out_type=jax.ShapeDtypeStruct((B, D), jnp.float32),
        mesh=mesh,
        scratch_types=[
            pltpu.VMEM((rpw,), jnp.int32),
            pltpu.VMEM((chunk, D), jnp.float32),
            pltpu.VMEM((chunk, D), jnp.float32),
            pltpu.SemaphoreType.DMA,
            pltpu.SemaphoreType.DMA,
        ],
    )
    def k(x_hbm, idx_hbm, out_hbm, idx_v, buf0, buf1, sem0, sem1):
        wid = lax.axis_index("s") * nc + lax.axis_index("c")
        base = wid * rpw
        pltpu.sync_copy(idx_hbm.at[pl.ds(base, rpw)], idx_v)

        bufs = (buf0, buf1)
        sems = (sem0, sem1)

        pltpu.async_copy(x_hbm.at[idx_v.at[pl.ds(0, chunk)]], buf0, sem0)

        def body(j, carry):
            slot = lax.rem(j, 2)

            def do(s):
                @pl.when(j + 1 < n_chunks)
                def _():
                    pltpu.async_copy(
                        x_hbm.at[idx_v.at[pl.ds((j + 1) * chunk, chunk)]],
                        bufs[1 - s],
                        sems[1 - s],
                    )

                pltpu.make_async_copy(
                    x_hbm.at[idx_v.at[pl.ds(j * chunk, chunk)]],
                    bufs[s],
                    sems[s],
                ).wait()
                pltpu.sync_copy(bufs[s], out_hbm.at[pl.ds(base + j * chunk, chunk)])

            @pl.when(slot == 0)
            def _():
                do(0)

            @pl.when(slot == 1)
            def _():
                do(1)

            return carry

        lax.fori_loop(0, n_chunks, body, 0)

    return k(x2, gidx)


def kernel(x):
    b, l, d = x.shape
    idx = _jitter_index(l)
    gidx = (jnp.arange(b, dtype=jnp.int32)[:, None] * l + idx[None, :]).reshape(-1)
    x2 = x.reshape(b * l, d)
    out = _sc_row_gather(x2, gidx, chunk=16)
    return out.reshape(b, l, d)
